# Initial kernel scaffold; baseline (speedup 1.0000x reference)
import jax
import jax.numpy as jnp
from jax.experimental import pallas as pl


def kernel(input, weight, bias, coordinates):
    raise NotImplementedError("write your pallas kernel here")


# speedup vs baseline: 5.8389x; 5.8389x over previous
import dataclasses
import functools

import jax
import jax.numpy as jnp
from jax import lax
from jax.experimental import pallas as pl
from jax.experimental.pallas import tpu as pltpu
from jax.experimental.pallas import tpu_sc as plsc

_NB = 4
_C = 512
_F = 128
_N = 256
_V = 100000
_CT = _NB * _C

_NUM_TILES = 32
_TOK_PER_TILE = _N // 16
_CHUNK = 2048
_NFULL = _V // _CHUNK
_TAIL = _V - _NFULL * _CHUNK
_TAILP = 1792
_VPAD = _NFULL * _CHUNK + _TAILP
_MINI = 16


def _logits_body(x_ref, w_ref, b_ref, out_ref):
    for b in range(_NB):
        xb = x_ref[:, b * _F:(b + 1) * _F]
        wb = w_ref[b]
        acc = lax.dot_general(
            xb, wb, (((1,), (1,)), ((), ())),
            preferred_element_type=jnp.float32)
        out_ref[:, b * _C:(b + 1) * _C] = acc + b_ref[b][None, :]


def _compute_logits(x, w, bias):
    return pl.pallas_call(
        _logits_body,
        out_shape=jax.ShapeDtypeStruct((_N, _CT), jnp.float32),
    )(x, w, bias)


def _decode_body(tt_hbm, idx_hbm, out_hbm, tail_hbm, slice_v, idx_v, out_v):
    cid = lax.axis_index("c")
    sid = lax.axis_index("s")
    n0 = sid * _TOK_PER_TILE

    pltpu.sync_copy(tt_hbm.at[pl.ds(n0, _TOK_PER_TILE), :], slice_v)

    def _do_minis(width):
        @pl.loop(0, width // _MINI)
        def _mini(j):
            idxs = [idx_v[b, pl.ds(j * _MINI, _MINI)] + (b * _C)
                    for b in range(_NB)]
            for n in range(_TOK_PER_TILE):
                rows = jnp.full((_MINI,), n, jnp.int32)
                acc = plsc.load_gather(slice_v, [rows, idxs[0]])
                for b in range(1, _NB):
                    acc = acc + plsc.load_gather(slice_v, [rows, idxs[b]])
                out_v[n, pl.ds(j * _MINI, _MINI)] = acc

    @pl.loop(cid, _NFULL, step=2)
    def _chunk(k):
        start = pl.multiple_of(k * _CHUNK, _CHUNK)
        pltpu.sync_copy(idx_hbm.at[:, pl.ds(start, _CHUNK)], idx_v)
        _do_minis(_CHUNK)
        pltpu.sync_copy(
            out_v,
            out_hbm.at[pl.ds(n0, _TOK_PER_TILE), pl.ds(start, _CHUNK)])

    @pl.when(cid == 1)
    def _tail():
        start = _NFULL * _CHUNK
        pltpu.sync_copy(idx_hbm.at[:, pl.ds(start, _TAILP)],
                        idx_v.at[:, pl.ds(0, _TAILP)])
        _do_minis(_TAILP)
        pltpu.sync_copy(out_v.at[:, pl.ds(0, _TAILP)],
                        tail_hbm.at[pl.ds(n0, _TOK_PER_TILE), :])


_SC_PARAMS = pltpu.CompilerParams()
if "needs_layout_passes" in pltpu.CompilerParams.__dataclass_fields__:
    _SC_PARAMS = dataclasses.replace(_SC_PARAMS, needs_layout_passes=False)


@functools.partial(
    pl.kernel,
    out_type=(jax.ShapeDtypeStruct((_N, _V), jnp.float32),
              jax.ShapeDtypeStruct((_N, _TAILP), jnp.float32)),
    compiler_params=_SC_PARAMS,
    mesh=plsc.VectorSubcoreMesh(core_axis_name="c", subcore_axis_name="s"),
    scratch_types=[
        pltpu.VMEM((_TOK_PER_TILE, _CT), jnp.float32),
        pltpu.VMEM((_NB, _CHUNK), jnp.int32),
        pltpu.VMEM((_TOK_PER_TILE, _CHUNK), jnp.float32),
    ],
)
def _decode(tt_hbm, idx_hbm, out_hbm, tail_hbm, slice_v, idx_v, out_v):
    _decode_body(tt_hbm, idx_hbm, out_hbm, tail_hbm, slice_v, idx_v, out_v)


@jax.jit
def kernel(input, weight, bias, coordinates):
    tt = _compute_logits(input, weight, bias)
    coords_pad = jnp.concatenate(
        [coordinates,
         jnp.zeros((_NB, _VPAD - _V), jnp.int32)], axis=1)
    out, tail = _decode(tt, coords_pad)
    return lax.dynamic_update_slice(out, tail[:, :_TAIL], (0, _NFULL * _CHUNK))


# speedup vs baseline: 6.1003x; 1.0448x over previous
import dataclasses
import functools

import jax
import jax.numpy as jnp
from jax import lax
from jax.experimental import pallas as pl
from jax.experimental.pallas import tpu as pltpu
from jax.experimental.pallas import tpu_sc as plsc

_NB = 4
_C = 512
_F = 128
_N = 256
_V = 100000
_CT = _NB * _C

_TOK_PER_TILE = _N // 16
_CHUNK = 2048
_NFULL = _V // _CHUNK
_PER_CORE = _NFULL // 2
_TAIL = _V - _NFULL * _CHUNK
_TAILP = 1792
_TAIL0 = 896
_TAIL1 = _TAIL - _TAIL0
_VPAD = _NFULL * _CHUNK + _TAILP
_MINI = 16


def _logits_body(x_ref, w_ref, b_ref, c_ref, out_ref, idx_ref):
    for b in range(_NB):
        xb = x_ref[:, b * _F:(b + 1) * _F]
        wb = w_ref[b]
        acc = lax.dot_general(
            xb, wb, (((1,), (1,)), ((), ())),
            preferred_element_type=jnp.float32)
        out_ref[:, b * _C:(b + 1) * _C] = acc + b_ref[b][None, :]
        idx_ref[b, :] = c_ref[b, :] + (b * _C)


def _compute_logits(x, w, bias, coords_pad):
    return pl.pallas_call(
        _logits_body,
        out_shape=(jax.ShapeDtypeStruct((_N, _CT), jnp.float32),
                   jax.ShapeDtypeStruct((_NB, _VPAD), jnp.int32)),
    )(x, w, bias, coords_pad)


def _decode_body(tt_hbm, idx_hbm, out_hbm,
                 slice_v, idx_v, out_v0, out_v1, tail_b, so0, so1):
    cid = lax.axis_index("c")
    sid = lax.axis_index("s")
    n0 = sid * _TOK_PER_TILE

    pltpu.sync_copy(tt_hbm.at[pl.ds(n0, _TOK_PER_TILE), :], slice_v)

    def _do_minis(out_v, width):
        @pl.loop(0, width // _MINI)
        def _mini(j):
            idxs = [idx_v[b, pl.ds(j * _MINI, _MINI)] for b in range(_NB)]
            for n in range(_TOK_PER_TILE):
                rows = jnp.full((_MINI,), n, jnp.int32)
                acc = plsc.load_gather(slice_v, [rows, idxs[0]])
                for b in range(1, _NB):
                    acc = acc + plsc.load_gather(slice_v, [rows, idxs[b]])
                out_v[n, pl.ds(j * _MINI, _MINI)] = acc

    bufs = (out_v0, out_v1)
    sems = (so0, so1)

    @pl.loop(0, _PER_CORE, step=2)
    def _chunk(i):
        for b in range(2):
            ii = i + b
            k = cid + 2 * ii
            start = pl.multiple_of(k * _CHUNK, _CHUNK)
            dst = out_hbm.at[pl.ds(n0, _TOK_PER_TILE), pl.ds(start, _CHUNK)]
            pltpu.sync_copy(idx_hbm.at[:, pl.ds(start, _CHUNK)], idx_v)

            @pl.when(ii >= 2)
            def _drain():
                pltpu.make_async_copy(bufs[b], dst, sems[b]).wait()

            _do_minis(bufs[b], _CHUNK)
            pltpu.async_copy(bufs[b], dst, sems[b])

    for b in range(2):
        dst = out_hbm.at[pl.ds(n0, _TOK_PER_TILE), pl.ds(0, _CHUNK)]
        pltpu.make_async_copy(bufs[b], dst, sems[b]).wait()

    tbase = _NFULL * _CHUNK

    @pl.when(cid == 0)
    def _tail0():
        pltpu.sync_copy(idx_hbm.at[:, pl.ds(tbase, _TAILP)],
                        idx_v.at[:, pl.ds(0, _TAILP)])
        _do_minis(out_v0, _TAIL0)
        pltpu.sync_copy(out_v0.at[:, pl.ds(0, _TAIL0)],
                        out_hbm.at[pl.ds(n0, _TOK_PER_TILE),
                                   pl.ds(tbase, _TAIL0)])

    @pl.when(cid == 1)
    def _tail1():
        pltpu.sync_copy(idx_hbm.at[:, pl.ds(tbase + _TAIL0, _TAILP - _TAIL0)],
                        idx_v.at[:, pl.ds(0, _TAILP - _TAIL0)])
        _do_minis(tail_b, _TAIL1)
        pltpu.sync_copy(tail_b,
                        out_hbm.at[pl.ds(n0, _TOK_PER_TILE),
                                   pl.ds(tbase + _TAIL0, _TAIL1)])


_SC_PARAMS = pltpu.CompilerParams()
if "needs_layout_passes" in pltpu.CompilerParams.__dataclass_fields__:
    _SC_PARAMS = dataclasses.replace(_SC_PARAMS, needs_layout_passes=False)


@functools.partial(
    pl.kernel,
    out_type=jax.ShapeDtypeStruct((_N, _V), jnp.float32),
    compiler_params=_SC_PARAMS,
    mesh=plsc.VectorSubcoreMesh(core_axis_name="c", subcore_axis_name="s"),
    scratch_types=[
        pltpu.VMEM((_TOK_PER_TILE, _CT), jnp.float32),
        pltpu.VMEM((_NB, _CHUNK), jnp.int32),
        pltpu.VMEM((_TOK_PER_TILE, _CHUNK), jnp.float32),
        pltpu.VMEM((_TOK_PER_TILE, _CHUNK), jnp.float32),
        pltpu.VMEM((_TOK_PER_TILE, _TAIL1), jnp.float32),
        pltpu.SemaphoreType.DMA,
        pltpu.SemaphoreType.DMA,
    ],
)
def _decode(tt_hbm, idx_hbm, out_hbm,
            slice_v, idx_v, out_v0, out_v1, tail_b, so0, so1):
    _decode_body(tt_hbm, idx_hbm, out_hbm,
                 slice_v, idx_v, out_v0, out_v1, tail_b, so0, so1)


@jax.jit
def kernel(input, weight, bias, coordinates):
    coords_pad = jnp.concatenate(
        [coordinates,
         jnp.zeros((_NB, _VPAD - _V), jnp.int32)], axis=1)
    tt, idxp = _compute_logits(input, weight, bias, coords_pad)
    return _decode(tt, idxp)
